# baseline (speedup 1.0000x reference)
import functools

import jax
import jax.numpy as jnp
from jax import lax
from jax.experimental import pallas as pl
from jax.experimental.pallas import tpu as pltpu

B = 1024
E = 32
W = 128
CH = 128
NCH = B // CH


def _tc_gather_dot(x0, x1, x0c, x1c, semb_t, sbias_t, femb_t, fbias_t):
    def body(x0_s, x1_s, x0c_v, x1c_v, semb_h, sbias_h, femb_h, fbias_h,
             d_ref, b_ref, se_sl, fe_sl, sb_sl, fb_sl,
             sem_se, sem_fe, sem_sb, sem_fb):
        lane3 = lax.broadcasted_iota(jnp.int32, (CH, 1, W), 2)

        def fire_chunk(c, p):
            def fire(j, carry):
                i = c * CH + j
                a0 = pl.multiple_of(x0_s[i] & ~(W - 1), W)
                a1 = pl.multiple_of(x1_s[i] & ~(W - 1), W)
                pltpu.make_async_copy(
                    semb_h.at[:, pl.ds(a0, W)], se_sl.at[p, j],
                    sem_se.at[p]).start()
                pltpu.make_async_copy(
                    femb_h.at[:, pl.ds(a1, W)], fe_sl.at[p, j],
                    sem_fe.at[p]).start()
                pltpu.make_async_copy(
                    sbias_h.at[:, pl.ds(a0, W)], sb_sl.at[p, j],
                    sem_sb.at[p]).start()
                pltpu.make_async_copy(
                    fbias_h.at[:, pl.ds(a1, W)], fb_sl.at[p, j],
                    sem_fb.at[p]).start()
                return carry

            lax.fori_loop(0, CH, fire, 0)

        fire_chunk(0, 0)
        for c in range(NCH):
            p = c % 2
            if c + 1 < NCH:
                fire_chunk(c + 1, (c + 1) % 2)

            def drain(j, carry, p=p):
                pltpu.make_async_copy(
                    semb_h.at[:, pl.ds(0, W)], se_sl.at[p, j],
                    sem_se.at[p]).wait()
                pltpu.make_async_copy(
                    femb_h.at[:, pl.ds(0, W)], fe_sl.at[p, j],
                    sem_fe.at[p]).wait()
                pltpu.make_async_copy(
                    sbias_h.at[:, pl.ds(0, W)], sb_sl.at[p, j],
                    sem_sb.at[p]).wait()
                pltpu.make_async_copy(
                    fbias_h.at[:, pl.ds(0, W)], fb_sl.at[p, j],
                    sem_fb.at[p]).wait()
                return carry

            lax.fori_loop(0, CH, drain, 0)

            sl = pl.ds(c * CH, CH)
            l0 = (x0c_v[sl] & (W - 1)).reshape(CH, 1, 1)
            l1 = (x1c_v[sl] & (W - 1)).reshape(CH, 1, 1)
            hot0 = (lane3 == l0).astype(jnp.float32)
            hot1 = (lane3 == l1).astype(jnp.float32)
            cols_a = jnp.sum(se_sl[p] * hot0, axis=2)
            cols_b = jnp.sum(fe_sl[p] * hot1, axis=2)
            d_ref[sl] = jnp.sum(cols_a * cols_b, axis=1,
                                keepdims=True) * (1.0 / E)
            sb = jnp.sum(sb_sl[p] * hot0, axis=2)
            fb = jnp.sum(fb_sl[p] * hot1, axis=2)
            b_ref[sl] = sb + fb

    return pl.pallas_call(
        body,
        in_specs=[
            pl.BlockSpec(memory_space=pltpu.SMEM),
            pl.BlockSpec(memory_space=pltpu.SMEM),
            pl.BlockSpec(memory_space=pltpu.VMEM),
            pl.BlockSpec(memory_space=pltpu.VMEM),
            pl.BlockSpec(memory_space=pltpu.MemorySpace.HBM),
            pl.BlockSpec(memory_space=pltpu.MemorySpace.HBM),
            pl.BlockSpec(memory_space=pltpu.MemorySpace.HBM),
            pl.BlockSpec(memory_space=pltpu.MemorySpace.HBM),
        ],
        out_specs=[
            pl.BlockSpec(memory_space=pltpu.VMEM),
            pl.BlockSpec(memory_space=pltpu.VMEM),
        ],
        out_shape=[
            jax.ShapeDtypeStruct((B, 1), jnp.float32),
            jax.ShapeDtypeStruct((B, 1), jnp.float32),
        ],
        scratch_shapes=[
            pltpu.VMEM((2, CH, E, W), jnp.float32),
            pltpu.VMEM((2, CH, E, W), jnp.float32),
            pltpu.VMEM((2, CH, 1, W), jnp.float32),
            pltpu.VMEM((2, CH, 1, W), jnp.float32),
            pltpu.SemaphoreType.DMA((2,)),
            pltpu.SemaphoreType.DMA((2,)),
            pltpu.SemaphoreType.DMA((2,)),
            pltpu.SemaphoreType.DMA((2,)),
        ],
    )(x0, x1, x0c, x1c, semb_t, sbias_t, femb_t, fbias_t)


def _tc_broadcast_sigmoid(d_row, b_col):
    def body(b_ref, d_ref, o_ref):
        s = b_ref[...] + d_ref[...]
        o_ref[...] = 1.0 / (1.0 + jnp.exp(-s))

    return pl.pallas_call(
        body,
        grid=(8,),
        in_specs=[
            pl.BlockSpec((B // 8, 1), lambda i: (i, 0)),
            pl.BlockSpec((1, B), lambda i: (0, 0)),
        ],
        out_specs=pl.BlockSpec((B // 8, B), lambda i: (i, 0)),
        out_shape=jax.ShapeDtypeStruct((B, B), jnp.float32),
    )(b_col, d_row)


def kernel(x, sample_embedding, sample_bias, feature_embedding, feature_bias):
    x0 = x[:, 0].astype(jnp.int32)
    x1 = x[:, 1].astype(jnp.int32)
    d_col, b_col = _tc_gather_dot(
        x0, x1, x0.reshape(B, 1), x1.reshape(B, 1),
        sample_embedding.T, sample_bias.T,
        feature_embedding.T, feature_bias.T,
    )
    return _tc_broadcast_sigmoid(d_col.reshape(1, B), b_col)


# speedup vs baseline: 1.0283x; 1.0283x over previous
import functools

import jax
import jax.numpy as jnp
from jax import lax
from jax.experimental import pallas as pl
from jax.experimental.pallas import tpu as pltpu

B = 1024
E = 32
W = 128
CH = 256
NCH = B // CH


def _tc_gather_dot(x0, x1, x0c, x1c, semb_t, sbias_t, femb_t, fbias_t):
    def body(x0_s, x1_s, x0c_v, x1c_v, semb_h, sbias_h, femb_h, fbias_h,
             d_ref, b_ref, se_sl, fe_sl, sb_sl, fb_sl,
             sem_se, sem_fe, sem_sb, sem_fb):
        lane3 = lax.broadcasted_iota(jnp.int32, (CH, 1, W), 2)

        for c in range(NCH):
            def fire(j, carry, c=c):
                i = c * CH + j
                a0 = pl.multiple_of(x0_s[i] & ~(W - 1), W)
                a1 = pl.multiple_of(x1_s[i] & ~(W - 1), W)
                pltpu.make_async_copy(
                    semb_h.at[:, pl.ds(a0, W)], se_sl.at[j], sem_se).start()
                pltpu.make_async_copy(
                    femb_h.at[:, pl.ds(a1, W)], fe_sl.at[j], sem_fe).start()
                pltpu.make_async_copy(
                    sbias_h.at[:, pl.ds(a0, W)], sb_sl.at[j], sem_sb).start()
                pltpu.make_async_copy(
                    fbias_h.at[:, pl.ds(a1, W)], fb_sl.at[j], sem_fb).start()
                return carry

            lax.fori_loop(0, CH, fire, 0)

            def drain(j, carry):
                pltpu.make_async_copy(
                    semb_h.at[:, pl.ds(0, W)], se_sl.at[j], sem_se).wait()
                pltpu.make_async_copy(
                    femb_h.at[:, pl.ds(0, W)], fe_sl.at[j], sem_fe).wait()
                pltpu.make_async_copy(
                    sbias_h.at[:, pl.ds(0, W)], sb_sl.at[j], sem_sb).wait()
                pltpu.make_async_copy(
                    fbias_h.at[:, pl.ds(0, W)], fb_sl.at[j], sem_fb).wait()
                return carry

            lax.fori_loop(0, CH, drain, 0)

            sl = pl.ds(c * CH, CH)
            l0 = (x0c_v[sl] & (W - 1)).reshape(CH, 1, 1)
            l1 = (x1c_v[sl] & (W - 1)).reshape(CH, 1, 1)
            hot0 = (lane3 == l0).astype(jnp.float32)
            hot1 = (lane3 == l1).astype(jnp.float32)
            cols_a = jnp.sum(se_sl[...] * hot0, axis=2)
            cols_b = jnp.sum(fe_sl[...] * hot1, axis=2)
            d_ref[sl] = jnp.sum(cols_a * cols_b, axis=1,
                                keepdims=True) * (1.0 / E)
            sb = jnp.sum(sb_sl[...] * hot0, axis=2)
            fb = jnp.sum(fb_sl[...] * hot1, axis=2)
            b_ref[sl] = sb + fb

    return pl.pallas_call(
        body,
        in_specs=[
            pl.BlockSpec(memory_space=pltpu.SMEM),
            pl.BlockSpec(memory_space=pltpu.SMEM),
            pl.BlockSpec(memory_space=pltpu.VMEM),
            pl.BlockSpec(memory_space=pltpu.VMEM),
            pl.BlockSpec(memory_space=pltpu.MemorySpace.HBM),
            pl.BlockSpec(memory_space=pltpu.MemorySpace.HBM),
            pl.BlockSpec(memory_space=pltpu.MemorySpace.HBM),
            pl.BlockSpec(memory_space=pltpu.MemorySpace.HBM),
        ],
        out_specs=[
            pl.BlockSpec(memory_space=pltpu.VMEM),
            pl.BlockSpec(memory_space=pltpu.VMEM),
        ],
        out_shape=[
            jax.ShapeDtypeStruct((B, 1), jnp.float32),
            jax.ShapeDtypeStruct((B, 1), jnp.float32),
        ],
        scratch_shapes=[
            pltpu.VMEM((CH, E, W), jnp.float32),
            pltpu.VMEM((CH, E, W), jnp.float32),
            pltpu.VMEM((CH, 1, W), jnp.float32),
            pltpu.VMEM((CH, 1, W), jnp.float32),
            pltpu.SemaphoreType.DMA,
            pltpu.SemaphoreType.DMA,
            pltpu.SemaphoreType.DMA,
            pltpu.SemaphoreType.DMA,
        ],
    )(x0, x1, x0c, x1c, semb_t, sbias_t, femb_t, fbias_t)


def _tc_broadcast_sigmoid(d_row, b_col):
    def body(b_ref, d_ref, o_ref):
        s = b_ref[...] + d_ref[...]
        o_ref[...] = 1.0 / (1.0 + jnp.exp(-s))

    return pl.pallas_call(
        body,
        grid=(8,),
        in_specs=[
            pl.BlockSpec((B // 8, 1), lambda i: (i, 0)),
            pl.BlockSpec((1, B), lambda i: (0, 0)),
        ],
        out_specs=pl.BlockSpec((B // 8, B), lambda i: (i, 0)),
        out_shape=jax.ShapeDtypeStruct((B, B), jnp.float32),
    )(b_col, d_row)


def kernel(x, sample_embedding, sample_bias, feature_embedding, feature_bias):
    x0 = x[:, 0].astype(jnp.int32)
    x1 = x[:, 1].astype(jnp.int32)
    d_col, b_col = _tc_gather_dot(
        x0, x1, x0.reshape(B, 1), x1.reshape(B, 1),
        sample_embedding.T, sample_bias.T,
        feature_embedding.T, feature_bias.T,
    )
    return _tc_broadcast_sigmoid(d_col.reshape(1, B), b_col)


# speedup vs baseline: 1.0355x; 1.0070x over previous
import functools

import jax
import jax.numpy as jnp
from jax import lax
from jax.experimental import pallas as pl
from jax.experimental.pallas import tpu as pltpu

B = 1024
E = 32
W = 128
CH = 512
NCH = B // CH


def _tc_gather_dot(x0, x1, x0c, x1c, semb_t, sbias_t, femb_t, fbias_t):
    def body(x0_s, x1_s, x0c_v, x1c_v, semb_h, sbias_h, femb_h, fbias_h,
             d_ref, b_ref, se_sl, fe_sl, sb_sl, fb_sl,
             sem_se, sem_fe, sem_sb, sem_fb):
        lane3 = lax.broadcasted_iota(jnp.int32, (CH, 1, W), 2)

        for c in range(NCH):
            def fire(j, carry, c=c):
                i = c * CH + j
                a0 = pl.multiple_of(x0_s[i] & ~(W - 1), W)
                a1 = pl.multiple_of(x1_s[i] & ~(W - 1), W)
                pltpu.make_async_copy(
                    semb_h.at[:, pl.ds(a0, W)], se_sl.at[j], sem_se).start()
                pltpu.make_async_copy(
                    femb_h.at[:, pl.ds(a1, W)], fe_sl.at[j], sem_fe).start()
                pltpu.make_async_copy(
                    sbias_h.at[:, pl.ds(a0, W)], sb_sl.at[j], sem_sb).start()
                pltpu.make_async_copy(
                    fbias_h.at[:, pl.ds(a1, W)], fb_sl.at[j], sem_fb).start()
                return carry

            lax.fori_loop(0, CH, fire, 0)

            def drain(j, carry):
                pltpu.make_async_copy(
                    semb_h.at[:, pl.ds(0, W)], se_sl.at[j], sem_se).wait()
                pltpu.make_async_copy(
                    femb_h.at[:, pl.ds(0, W)], fe_sl.at[j], sem_fe).wait()
                pltpu.make_async_copy(
                    sbias_h.at[:, pl.ds(0, W)], sb_sl.at[j], sem_sb).wait()
                pltpu.make_async_copy(
                    fbias_h.at[:, pl.ds(0, W)], fb_sl.at[j], sem_fb).wait()
                return carry

            lax.fori_loop(0, CH, drain, 0)

            sl = pl.ds(c * CH, CH)
            l0 = (x0c_v[sl] & (W - 1)).reshape(CH, 1, 1)
            l1 = (x1c_v[sl] & (W - 1)).reshape(CH, 1, 1)
            hot0 = (lane3 == l0).astype(jnp.float32)
            hot1 = (lane3 == l1).astype(jnp.float32)
            cols_a = jnp.sum(se_sl[...] * hot0, axis=2)
            cols_b = jnp.sum(fe_sl[...] * hot1, axis=2)
            d_ref[sl] = jnp.sum(cols_a * cols_b, axis=1,
                                keepdims=True) * (1.0 / E)
            sb = jnp.sum(sb_sl[...] * hot0, axis=2)
            fb = jnp.sum(fb_sl[...] * hot1, axis=2)
            b_ref[sl] = sb + fb

    return pl.pallas_call(
        body,
        in_specs=[
            pl.BlockSpec(memory_space=pltpu.SMEM),
            pl.BlockSpec(memory_space=pltpu.SMEM),
            pl.BlockSpec(memory_space=pltpu.VMEM),
            pl.BlockSpec(memory_space=pltpu.VMEM),
            pl.BlockSpec(memory_space=pltpu.MemorySpace.HBM),
            pl.BlockSpec(memory_space=pltpu.MemorySpace.HBM),
            pl.BlockSpec(memory_space=pltpu.MemorySpace.HBM),
            pl.BlockSpec(memory_space=pltpu.MemorySpace.HBM),
        ],
        out_specs=[
            pl.BlockSpec(memory_space=pltpu.VMEM),
            pl.BlockSpec(memory_space=pltpu.VMEM),
        ],
        out_shape=[
            jax.ShapeDtypeStruct((B, 1), jnp.float32),
            jax.ShapeDtypeStruct((B, 1), jnp.float32),
        ],
        scratch_shapes=[
            pltpu.VMEM((CH, E, W), jnp.float32),
            pltpu.VMEM((CH, E, W), jnp.float32),
            pltpu.VMEM((CH, 1, W), jnp.float32),
            pltpu.VMEM((CH, 1, W), jnp.float32),
            pltpu.SemaphoreType.DMA,
            pltpu.SemaphoreType.DMA,
            pltpu.SemaphoreType.DMA,
            pltpu.SemaphoreType.DMA,
        ],
    )(x0, x1, x0c, x1c, semb_t, sbias_t, femb_t, fbias_t)


def _tc_broadcast_sigmoid(d_row, b_col):
    def body(b_ref, d_ref, o_ref):
        s = b_ref[...] + d_ref[...]
        o_ref[...] = 1.0 / (1.0 + jnp.exp(-s))

    return pl.pallas_call(
        body,
        grid=(8,),
        in_specs=[
            pl.BlockSpec((B // 8, 1), lambda i: (i, 0)),
            pl.BlockSpec((1, B), lambda i: (0, 0)),
        ],
        out_specs=pl.BlockSpec((B // 8, B), lambda i: (i, 0)),
        out_shape=jax.ShapeDtypeStruct((B, B), jnp.float32),
    )(b_col, d_row)


def kernel(x, sample_embedding, sample_bias, feature_embedding, feature_bias):
    x0 = x[:, 0].astype(jnp.int32)
    x1 = x[:, 1].astype(jnp.int32)
    d_col, b_col = _tc_gather_dot(
        x0, x1, x0.reshape(B, 1), x1.reshape(B, 1),
        sample_embedding.T, sample_bias.T,
        feature_embedding.T, feature_bias.T,
    )
    return _tc_broadcast_sigmoid(d_col.reshape(1, B), b_col)
